# baseline (speedup 1.0000x reference)
import jax
import jax.numpy as jnp
from jax import lax
from jax.experimental import pallas as pl
from jax.experimental.pallas import tpu as pltpu
from jax.experimental.pallas import tpu_sc as plsc

V = 50000
F = 128
NUM_E = 2048
Q = 64
NCHUNK = 4
CHUNK = 12544
NPAD = NCHUNK * CHUNK
DUMMY = NPAD - 1
NSUB = 16
NWORK = 32
TPT = (NUM_E * Q) // NSUB
IBLK = 2048
EPB = NUM_E // NWORK
BATCH = 48
NBUF = 3
STRIPE = CHUNK // NSUB
CBUF = TPT + NBUF * BATCH
LOC_PAD = CHUNK
LOC_MASK = (1 << 14) - 1


def _dedup_body(he_ref, out_ref):
    he = he_ref[...]
    blk = he.shape[0]
    q_iota = lax.broadcasted_iota(jnp.int32, (blk, Q), 1)
    dup = jnp.zeros((blk, Q), jnp.bool_)
    for d in range(1, Q):
        shifted = lax.pad(he[:, :Q - d], jnp.int32(-1),
                          ((0, 0, 0), (d, 0, 0)))
        dup = dup | ((he == shifted) & (q_iota >= d))
    out_ref[...] = jnp.where(dup, DUMMY, he)


def _gather_mean_body(he_hbm, x_hbm, idxm_hbm, agg_hbm, deg_hbm,
                      idx_v, rows_a, rows_b, blk_v, hist_v, sem_a, sem_b):
    wid = lax.axis_index("s") * 2 + lax.axis_index("c")
    e0 = wid * EPB
    pltpu.sync_copy(he_hbm.at[pl.ds(e0 * Q, EPB * Q)], idx_v)

    def start(e, rows, sem):
        pltpu.async_copy(x_hbm.at[idx_v.at[pl.ds(e * Q, Q)]], rows, sem)

    def wait(rows, sem):
        pltpu.make_async_copy(x_hbm.at[pl.ds(0, Q)], rows, sem).wait()

    def reduce_to(rows, e):
        def rbody(r, acc):
            return tuple(acc[j] + rows[r, pl.ds(j * 16, 16)]
                         for j in range(8))

        acc = lax.fori_loop(
            0, Q, rbody,
            tuple(jnp.zeros((16,), jnp.float32) for _ in range(8)))
        for j in range(8):
            blk_v[e, pl.ds(j * 16, 16)] = acc[j] * (1.0 / Q)

    start(0, rows_a, sem_a)

    @pl.loop(0, EPB, step=2)
    def _(e):
        start(e + 1, rows_b, sem_b)
        wait(rows_a, sem_a)
        reduce_to(rows_a, e)

        @pl.when(e + 2 < EPB)
        def _():
            start(e + 2, rows_a, sem_a)

        wait(rows_b, sem_b)
        reduce_to(rows_b, e + 1)

    pltpu.sync_copy(blk_v, agg_hbm.at[pl.ds(e0, EPB)])

    @pl.loop(0, NPAD // 16)
    def _(v):
        hist_v[pl.ds(v * 16, 16)] = jnp.zeros((16,), jnp.float32)

    pltpu.sync_copy(idxm_hbm.at[pl.ds(e0 * Q, EPB * Q)], idx_v)
    ones16 = jnp.ones((16,), jnp.float32)

    @pl.loop(0, (EPB * Q) // 16)
    def _(v):
        iv = idx_v[pl.ds(v * 16, 16)]
        plsc.addupdate_scatter(hist_v, [iv], ones16)

    pltpu.sync_copy(hist_v, deg_hbm.at[wid])


def _scatter_body(idx_hbm, agg_hbm, zero_hbm, y0_hbm,
                  idx_v, pk_v, loc_st, eid_st, rows_v, acc_sh,
                  gsem, ssem):
    cid = lax.axis_index("c")
    t = lax.axis_index("s")
    lane = lax.iota(jnp.int32, 16)

    for ci in range(NCHUNK // 2):
        base = ((NCHUNK // 2) * cid + ci) * CHUNK
        pltpu.sync_copy(zero_hbm, acc_sh.at[pl.ds(t * STRIPE, STRIPE)])

        @pl.loop(0, CBUF // 16)
        def _(v):
            pk_v[pl.ds(v * 16, 16)] = jnp.full((16,), LOC_PAD, jnp.int32)

        plsc.subcore_barrier()

        def blk_scan(bi, off):
            pltpu.sync_copy(idx_hbm.at[pl.ds(t * TPT + bi * IBLK, IBLK)],
                            idx_v)

            def sbody(v, off):
                iv = idx_v[pl.ds(v * 16, 16)]
                loc = iv - base
                m = (loc >= 0) & (loc < CHUNK)
                g = t * TPT + bi * IBLK + v * 16 + lane
                eid = g // Q
                packed = loc | (eid << 14)
                plsc.store_compressed(pk_v.at[pl.ds(off, 16)], packed,
                                      mask=m)
                return off + jnp.max(plsc.all_reduce_population_count(m))

            return lax.fori_loop(0, IBLK // 16, sbody, off)

        n = lax.fori_loop(0, TPT // IBLK, blk_scan, jnp.int32(0))
        nb = (n + (BATCH - 1)) // BATCH
        nq = (nb + (NBUF - 1)) // NBUF

        def unpack(i, k):
            for j in range(BATCH // 16):
                p = pk_v[pl.ds(i * BATCH + j * 16, 16)]
                loc_st[k, pl.ds(j * 16, 16)] = p & LOC_MASK
                eid_st[k, pl.ds(j * 16, 16)] = lax.shift_right_logical(
                    p, 14)

        def qbody(qi, carry):
            for k in range(NBUF):
                @pl.when(qi > 0)
                def _():
                    pltpu.make_async_copy(
                        rows_v.at[k], acc_sh.at[pl.ds(0, BATCH)],
                        ssem.at[k]).wait()
                unpack(qi * NBUF + k, k)
                pltpu.async_copy(agg_hbm.at[eid_st.at[k]], rows_v.at[k],
                                 gsem.at[k])
            for k in range(NBUF):
                pltpu.make_async_copy(
                    agg_hbm.at[pl.ds(0, BATCH)], rows_v.at[k],
                    gsem.at[k]).wait()
                pltpu.async_copy(rows_v.at[k], acc_sh.at[loc_st.at[k]],
                                 ssem.at[k], add=True)
            return carry

        lax.fori_loop(0, nq, qbody, jnp.int32(0))
        for k in range(NBUF):
            @pl.when(nq > 0)
            def _():
                pltpu.make_async_copy(
                    rows_v.at[k], acc_sh.at[pl.ds(0, BATCH)],
                    ssem.at[k]).wait()

        plsc.subcore_barrier()
        pltpu.sync_copy(acc_sh.at[pl.ds(t * STRIPE, STRIPE)],
                        y0_hbm.at[pl.ds(base + t * STRIPE, STRIPE)])
        plsc.subcore_barrier()


def _norm_body(y_ref, d_ref, w_ref, b_ref, out_ref):
    y = y_ref[...]
    deg = jnp.sum(d_ref[...], axis=1, keepdims=True)
    deg = jnp.maximum(deg, 1.0)
    xn = (y / deg).astype(jnp.bfloat16)
    out = lax.dot_general(xn, w_ref[...].astype(jnp.bfloat16),
                          dimension_numbers=(((1,), (1,)), ((), ())),
                          preferred_element_type=jnp.float32)
    out_ref[...] = (out + b_ref[...])[None]


def kernel(X, hyperedges, W, b):
    x2 = X.reshape(V, F)
    he_flat = hyperedges.reshape(-1)

    idxm = pl.pallas_call(
        _dedup_body,
        grid=(8,),
        in_specs=[pl.BlockSpec((NUM_E // 8, Q), lambda i: (i, 0))],
        out_specs=pl.BlockSpec((NUM_E // 8, Q), lambda i: (i, 0)),
        out_shape=jax.ShapeDtypeStruct((NUM_E, Q), jnp.int32),
    )(hyperedges)
    idxm_flat = idxm.reshape(-1)

    agg, deg_part = pl.kernel(
        _gather_mean_body,
        out_type=[jax.ShapeDtypeStruct((NUM_E, F), jnp.float32),
                  jax.ShapeDtypeStruct((NWORK, NPAD), jnp.float32)],
        mesh=plsc.VectorSubcoreMesh(core_axis_name="c", subcore_axis_name="s"),
        compiler_params=pltpu.CompilerParams(needs_layout_passes=False),
        scratch_types=[pltpu.VMEM((EPB * Q,), jnp.int32),
                       pltpu.VMEM((Q, F), jnp.float32),
                       pltpu.VMEM((Q, F), jnp.float32),
                       pltpu.VMEM((EPB, F), jnp.float32),
                       pltpu.VMEM((NPAD,), jnp.float32),
                       pltpu.SemaphoreType.DMA,
                       pltpu.SemaphoreType.DMA],
    )(he_flat, x2, idxm_flat)

    y0 = pl.kernel(
        _scatter_body,
        out_type=jax.ShapeDtypeStruct((NPAD, F), jnp.float32),
        mesh=plsc.VectorSubcoreMesh(core_axis_name="c", subcore_axis_name="s"),
        compiler_params=pltpu.CompilerParams(needs_layout_passes=False),
        scratch_types=[pltpu.VMEM((IBLK,), jnp.int32),
                       pltpu.VMEM((CBUF,), jnp.int32),
                       pltpu.VMEM((NBUF, BATCH), jnp.int32),
                       pltpu.VMEM((NBUF, BATCH), jnp.int32),
                       pltpu.VMEM((NBUF, BATCH, F), jnp.float32),
                       pltpu.VMEM_SHARED((CHUNK + 1, F), jnp.float32),
                       pltpu.SemaphoreType.DMA((NBUF,)),
                       pltpu.SemaphoreType.DMA((NBUF,))],
    )(idxm_flat, agg, jnp.zeros((STRIPE, F), jnp.float32))

    out = pl.pallas_call(
        _norm_body,
        grid=(NPAD // 1024,),
        in_specs=[pl.BlockSpec((1024, F), lambda i: (i, 0)),
                  pl.BlockSpec((1024, NWORK), lambda i: (i, 0)),
                  pl.BlockSpec((F, F), lambda i: (0, 0)),
                  pl.BlockSpec((1, F), lambda i: (0, 0))],
        out_specs=pl.BlockSpec((1, 1024, F), lambda i: (0, i, 0)),
        out_shape=jax.ShapeDtypeStruct((1, V, F), jnp.float32),
    )(y0, deg_part.T, W, b.reshape(1, F))
    return out


# speedup vs baseline: 1.0643x; 1.0643x over previous
import jax
import jax.numpy as jnp
from jax import lax
from jax.experimental import pallas as pl
from jax.experimental.pallas import tpu as pltpu
from jax.experimental.pallas import tpu_sc as plsc

V = 50000
F = 128
NUM_E = 2048
Q = 64
NCHUNK = 4
CHUNK = 12544
NPAD = NCHUNK * CHUNK
DUMMY = NPAD - 1
NSUB = 16
NWORK = 32
TPT = (NUM_E * Q) // NSUB
IBLK = 2048
EPB = NUM_E // NWORK
BATCH = 64
NBUF = 2
STRIPE = CHUNK // NSUB
CBUF = TPT + NBUF * BATCH
LOC_PAD = CHUNK
LOC_MASK = (1 << 14) - 1


def _dedup_body(he_ref, out_ref):
    he = he_ref[...]
    blk = he.shape[0]
    q_iota = lax.broadcasted_iota(jnp.int32, (blk, Q), 1)
    dup = jnp.zeros((blk, Q), jnp.bool_)
    for d in range(1, Q):
        shifted = lax.pad(he[:, :Q - d], jnp.int32(-1),
                          ((0, 0, 0), (d, 0, 0)))
        dup = dup | ((he == shifted) & (q_iota >= d))
    out_ref[...] = jnp.where(dup, DUMMY, he)


def _gather_mean_body(he_hbm, x_hbm, idxm_hbm, agg_hbm, deg_hbm,
                      idx_v, rows_a, rows_b, blk_v, hist_v, sem_a, sem_b):
    wid = lax.axis_index("s") * 2 + lax.axis_index("c")
    e0 = wid * EPB
    pltpu.sync_copy(he_hbm.at[pl.ds(e0 * Q, EPB * Q)], idx_v)

    def start(e, rows, sem):
        pltpu.async_copy(x_hbm.at[idx_v.at[pl.ds(e * Q, Q)]], rows, sem)

    def wait(rows, sem):
        pltpu.make_async_copy(x_hbm.at[pl.ds(0, Q)], rows, sem).wait()

    def reduce_to(rows, e):
        def rbody(r, acc):
            return tuple(acc[j] + rows[r, pl.ds(j * 16, 16)]
                         for j in range(8))

        acc = lax.fori_loop(
            0, Q, rbody,
            tuple(jnp.zeros((16,), jnp.float32) for _ in range(8)))
        for j in range(8):
            blk_v[e, pl.ds(j * 16, 16)] = acc[j] * (1.0 / Q)

    start(0, rows_a, sem_a)

    @pl.loop(0, EPB, step=2)
    def _(e):
        start(e + 1, rows_b, sem_b)
        wait(rows_a, sem_a)
        reduce_to(rows_a, e)

        @pl.when(e + 2 < EPB)
        def _():
            start(e + 2, rows_a, sem_a)

        wait(rows_b, sem_b)
        reduce_to(rows_b, e + 1)

    pltpu.sync_copy(blk_v, agg_hbm.at[pl.ds(e0, EPB)])

    @pl.loop(0, NPAD // 16)
    def _(v):
        hist_v[pl.ds(v * 16, 16)] = jnp.zeros((16,), jnp.float32)

    pltpu.sync_copy(idxm_hbm.at[pl.ds(e0 * Q, EPB * Q)], idx_v)
    ones16 = jnp.ones((16,), jnp.float32)

    @pl.loop(0, (EPB * Q) // 16)
    def _(v):
        iv = idx_v[pl.ds(v * 16, 16)]
        plsc.addupdate_scatter(hist_v, [iv], ones16)

    pltpu.sync_copy(hist_v, deg_hbm.at[wid])


def _scatter_body(idx_hbm, agg_hbm, zero_hbm, y0_hbm,
                  idx_v, pk_v, loc_st, eid_st, rows_v, acc_sh,
                  gsem, ssem):
    cid = lax.axis_index("c")
    t = lax.axis_index("s")
    lane = lax.iota(jnp.int32, 16)

    for ci in range(NCHUNK // 2):
        base = ((NCHUNK // 2) * cid + ci) * CHUNK
        pltpu.sync_copy(zero_hbm, acc_sh.at[pl.ds(t * STRIPE, STRIPE)])

        @pl.loop(0, CBUF // 16)
        def _(v):
            pk_v[pl.ds(v * 16, 16)] = jnp.full((16,), LOC_PAD, jnp.int32)

        plsc.subcore_barrier()

        def blk_scan(bi, off):
            pltpu.sync_copy(idx_hbm.at[pl.ds(t * TPT + bi * IBLK, IBLK)],
                            idx_v)

            def sbody(v, off):
                iv = idx_v[pl.ds(v * 16, 16)]
                loc = iv - base
                m = (loc >= 0) & (loc < CHUNK)
                g = t * TPT + bi * IBLK + v * 16 + lane
                eid = g // Q
                packed = loc | (eid << 14)
                plsc.store_compressed(pk_v.at[pl.ds(off, 16)], packed,
                                      mask=m)
                return off + jnp.max(plsc.all_reduce_population_count(m))

            return lax.fori_loop(0, IBLK // 16, sbody, off)

        n = lax.fori_loop(0, TPT // IBLK, blk_scan, jnp.int32(0))
        nb = (n + (BATCH - 1)) // BATCH
        nq = (nb + (NBUF - 1)) // NBUF

        def unpack(i, k):
            for j in range(BATCH // 16):
                p = pk_v[pl.ds(i * BATCH + j * 16, 16)]
                loc_st[k, pl.ds(j * 16, 16)] = p & LOC_MASK
                eid_st[k, pl.ds(j * 16, 16)] = lax.shift_right_logical(
                    p, 14)

        def qbody(qi, carry):
            for k in range(NBUF):
                @pl.when(qi > 0)
                def _():
                    pltpu.make_async_copy(
                        rows_v.at[k], acc_sh.at[pl.ds(0, BATCH)],
                        ssem.at[k]).wait()
                unpack(qi * NBUF + k, k)
                pltpu.async_copy(agg_hbm.at[eid_st.at[k]], rows_v.at[k],
                                 gsem.at[k])
            for k in range(NBUF):
                pltpu.make_async_copy(
                    agg_hbm.at[pl.ds(0, BATCH)], rows_v.at[k],
                    gsem.at[k]).wait()
                pltpu.async_copy(rows_v.at[k], acc_sh.at[loc_st.at[k]],
                                 ssem.at[k], add=True)
            return carry

        lax.fori_loop(0, nq, qbody, jnp.int32(0))
        for k in range(NBUF):
            @pl.when(nq > 0)
            def _():
                pltpu.make_async_copy(
                    rows_v.at[k], acc_sh.at[pl.ds(0, BATCH)],
                    ssem.at[k]).wait()

        plsc.subcore_barrier()
        pltpu.sync_copy(acc_sh.at[pl.ds(t * STRIPE, STRIPE)],
                        y0_hbm.at[pl.ds(base + t * STRIPE, STRIPE)])
        plsc.subcore_barrier()


def _norm_body(y_ref, d_ref, w_ref, b_ref, out_ref):
    y = y_ref[...]
    deg = jnp.sum(d_ref[...], axis=1, keepdims=True)
    deg = jnp.maximum(deg, 1.0)
    xn = (y / deg).astype(jnp.bfloat16)
    out = lax.dot_general(xn, w_ref[...].astype(jnp.bfloat16),
                          dimension_numbers=(((1,), (1,)), ((), ())),
                          preferred_element_type=jnp.float32)
    out_ref[...] = (out + b_ref[...])[None]


def kernel(X, hyperedges, W, b):
    x2 = X.reshape(V, F)
    he_flat = hyperedges.reshape(-1)

    idxm = pl.pallas_call(
        _dedup_body,
        grid=(8,),
        in_specs=[pl.BlockSpec((NUM_E // 8, Q), lambda i: (i, 0))],
        out_specs=pl.BlockSpec((NUM_E // 8, Q), lambda i: (i, 0)),
        out_shape=jax.ShapeDtypeStruct((NUM_E, Q), jnp.int32),
    )(hyperedges)
    idxm_flat = idxm.reshape(-1)

    agg, deg_part = pl.kernel(
        _gather_mean_body,
        out_type=[jax.ShapeDtypeStruct((NUM_E, F), jnp.float32),
                  jax.ShapeDtypeStruct((NWORK, NPAD), jnp.float32)],
        mesh=plsc.VectorSubcoreMesh(core_axis_name="c", subcore_axis_name="s"),
        compiler_params=pltpu.CompilerParams(needs_layout_passes=False),
        scratch_types=[pltpu.VMEM((EPB * Q,), jnp.int32),
                       pltpu.VMEM((Q, F), jnp.float32),
                       pltpu.VMEM((Q, F), jnp.float32),
                       pltpu.VMEM((EPB, F), jnp.float32),
                       pltpu.VMEM((NPAD,), jnp.float32),
                       pltpu.SemaphoreType.DMA,
                       pltpu.SemaphoreType.DMA],
    )(he_flat, x2, idxm_flat)

    y0 = pl.kernel(
        _scatter_body,
        out_type=jax.ShapeDtypeStruct((NPAD, F), jnp.float32),
        mesh=plsc.VectorSubcoreMesh(core_axis_name="c", subcore_axis_name="s"),
        compiler_params=pltpu.CompilerParams(needs_layout_passes=False),
        scratch_types=[pltpu.VMEM((IBLK,), jnp.int32),
                       pltpu.VMEM((CBUF,), jnp.int32),
                       pltpu.VMEM((NBUF, BATCH), jnp.int32),
                       pltpu.VMEM((NBUF, BATCH), jnp.int32),
                       pltpu.VMEM((NBUF, BATCH, F), jnp.float32),
                       pltpu.VMEM_SHARED((CHUNK + 1, F), jnp.float32),
                       pltpu.SemaphoreType.DMA((NBUF,)),
                       pltpu.SemaphoreType.DMA((NBUF,))],
    )(idxm_flat, agg, jnp.zeros((STRIPE, F), jnp.float32))

    out = pl.pallas_call(
        _norm_body,
        grid=(NPAD // 1024,),
        in_specs=[pl.BlockSpec((1024, F), lambda i: (i, 0)),
                  pl.BlockSpec((1024, NWORK), lambda i: (i, 0)),
                  pl.BlockSpec((F, F), lambda i: (0, 0)),
                  pl.BlockSpec((1, F), lambda i: (0, 0))],
        out_specs=pl.BlockSpec((1, 1024, F), lambda i: (0, i, 0)),
        out_shape=jax.ShapeDtypeStruct((1, V, F), jnp.float32),
    )(y0, deg_part.T, W, b.reshape(1, F))
    return out
